# baseline (speedup 1.0000x reference)
import functools

import jax
import jax.numpy as jnp
from jax import lax
from jax.experimental import pallas as pl
from jax.experimental.pallas import tpu as pltpu
from jax.experimental.pallas import tpu_sc as plsc

D = 64
R = 8
SCALING = 2.0
NC = 2
NS = 16
NW = NC * NS
L = 16

SG = 128

_SC_PARAMS = pltpu.CompilerParams(use_tc_tiling_on_sc=False,
                                  needs_layout_passes=False)


@functools.lru_cache(maxsize=None)
def _build_padder():
    NE = 1000000
    BC = 1024

    def body(wt_ref, out_ref):
        b = wt_ref[...]
        bp = jnp.concatenate([b, jnp.zeros_like(b)], axis=0)
        out_ref[...] = jnp.transpose(bp)

    import math
    grid = (math.ceil(NE / BC),)
    return pl.pallas_call(
        body,
        grid=grid,
        in_specs=[pl.BlockSpec((D, BC), lambda j: (0, j))],
        out_specs=pl.BlockSpec((BC, 2 * D), lambda j: (j, 0)),
        out_shape=jax.ShapeDtypeStruct((NE, 2 * D), jnp.float32),
    )


@functools.lru_cache(maxsize=None)
def _build_fast(n_total):
    n_per_w = n_total // NW
    CH = 256
    n_chunks = n_per_w // CH
    n_sub = CH // SG

    mesh = plsc.VectorSubcoreMesh(core_axis_name="c", subcore_axis_name="s")

    @functools.partial(
        pl.kernel,
        mesh=mesh,
        out_type=jax.ShapeDtypeStruct((n_total, D), jnp.float32),
        scratch_types=[
            pltpu.VMEM((n_per_w,), jnp.int32),
            pltpu.VMEM((CH, 2 * D), jnp.float32),
            pltpu.VMEM((CH, D), jnp.float32),
            pltpu.SemaphoreType.DMA,
        ],
        compiler_params=_SC_PARAMS,
    )
    def k(ids_hbm, wp_hbm, out_hbm, idx_v, pairbuf, obuf, sem):
        cid = lax.axis_index("c")
        sid = lax.axis_index("s")
        wid = sid * NC + cid
        base = wid * n_per_w
        pltpu.sync_copy(ids_hbm.at[pl.ds(base, n_per_w)], idx_v)

        lane = lax.iota(jnp.int32, L)

        def chunk_body(kk, carry):
            cbase = kk * CH
            copies = []
            for j in range(n_sub):
                isl = idx_v.at[pl.ds(cbase + j * SG, SG)]
                copies.append(pltpu.async_copy(
                    wp_hbm.at[isl], pairbuf.at[pl.ds(j * SG, SG)], sem))
            for cp in copies:
                cp.wait()

            def row_body(rr, c2):
                full_r = jnp.full((L,), rr, jnp.int32)
                for c in range(D // L):
                    cols = c * L + lane
                    v = plsc.load_gather(pairbuf, [full_r, cols])
                    plsc.store_scatter(obuf, [full_r, cols], v)
                return c2

            lax.fori_loop(0, CH, row_body, 0)

            pltpu.sync_copy(obuf, out_hbm.at[pl.ds(base + cbase, CH)])
            return carry

        lax.fori_loop(0, n_chunks, chunk_body, 0)

    return k


@functools.lru_cache(maxsize=None)
def _build_slow(n_total):
    n_per_w = n_total // NW
    CH = 640
    n_chunks = n_per_w // CH
    n_sub = CH // SG

    mesh = plsc.VectorSubcoreMesh(core_axis_name="c", subcore_axis_name="s")

    @functools.partial(
        pl.kernel,
        mesh=mesh,
        out_type=jax.ShapeDtypeStruct((n_total, D), jnp.float32),
        scratch_types=[
            pltpu.VMEM((n_per_w,), jnp.int32),
            pltpu.VMEM((CH, D), jnp.float32),
            pltpu.VMEM((CH, R), jnp.float32),
            pltpu.VMEM((R, D), jnp.float32),
            pltpu.SemaphoreType.DMA,
            pltpu.SemaphoreType.DMA,
        ],
        compiler_params=_SC_PARAMS,
    )
    def k(ids_hbm, w_hbm, a_hbm, b_hbm, out_hbm,
          idx_all, wbuf, bbuf, abuf, semw, semb):
        cid = lax.axis_index("c")
        sid = lax.axis_index("s")
        wid = sid * NC + cid
        base = wid * n_per_w
        pltpu.sync_copy(ids_hbm.at[pl.ds(base, n_per_w)], idx_all)
        pltpu.sync_copy(a_hbm, abuf)

        lane = lax.iota(jnp.int32, L)

        def chunk_body(kk, carry):
            cbase = kk * CH
            copies = []
            for j in range(n_sub):
                isl = idx_all.at[pl.ds(cbase + j * SG, SG)]
                copies.append(pltpu.async_copy(
                    w_hbm.at[isl], wbuf.at[pl.ds(j * SG, SG)], semw))
                copies.append(pltpu.async_copy(
                    b_hbm.at[isl], bbuf.at[pl.ds(j * SG, SG)], semb))
            for cp in copies:
                cp.wait()

            def row_body(rr, c2):
                full_r = jnp.full((L,), rr, jnp.int32)
                for c in range(D // L):
                    cols = c * L + lane
                    acc = plsc.load_gather(wbuf, [full_r, cols])
                    for r in range(R):
                        bv = plsc.load_gather(
                            bbuf, [full_r, jnp.full((L,), r, jnp.int32)])
                        av = abuf[r, pl.ds(c * L, L)]
                        acc = acc + (bv * SCALING) * av
                    plsc.store_scatter(wbuf, [full_r, cols], acc)
                return c2

            lax.fori_loop(0, CH, row_body, 0)

            pltpu.sync_copy(wbuf, out_hbm.at[pl.ds(base + cbase, CH)])
            return carry

        lax.fori_loop(0, n_chunks, chunk_body, 0)

    return k


def kernel(input_ids, weight, lora_A, lora_B):
    n_total = input_ids.shape[0] * input_ids.shape[1]
    ids = input_ids.reshape(n_total).astype(jnp.int32)
    wp = _build_padder()(weight.T)
    any_nz = jnp.any(lora_B != 0)

    def fast():
        return _build_fast(n_total)(ids, wp)

    def slow():
        return _build_slow(n_total)(ids, weight, lora_A, lora_B)

    out = lax.cond(any_nz, slow, fast)
    return out.reshape(input_ids.shape + (D,))


# speedup vs baseline: 1.1356x; 1.1356x over previous
import functools

import jax
import jax.numpy as jnp
from jax import lax
from jax.experimental import pallas as pl
from jax.experimental.pallas import tpu as pltpu
from jax.experimental.pallas import tpu_sc as plsc

D = 64
R = 8
SCALING = 2.0
NC = 2
NS = 16
NW = NC * NS
L = 16

SG = 128

_SC_PARAMS = pltpu.CompilerParams(use_tc_tiling_on_sc=False,
                                  needs_layout_passes=False)


@functools.lru_cache(maxsize=None)
def _build_fast(n_total):
    n_per_w = n_total // NW
    CH = 640
    n_chunks = n_per_w // CH
    n_sub = CH // SG

    mesh = plsc.VectorSubcoreMesh(core_axis_name="c", subcore_axis_name="s")

    @functools.partial(
        pl.kernel,
        mesh=mesh,
        out_type=jax.ShapeDtypeStruct((n_total, D), jnp.float32),
        scratch_types=[
            pltpu.VMEM((n_per_w,), jnp.int32),
            pltpu.VMEM((CH, D), jnp.float32),
            pltpu.SemaphoreType.DMA,
        ],
        compiler_params=_SC_PARAMS,
    )
    def k(ids_hbm, w_hbm, out_hbm, idx_v, wbuf, sem):
        cid = lax.axis_index("c")
        sid = lax.axis_index("s")
        wid = sid * NC + cid
        base = wid * n_per_w
        pltpu.sync_copy(ids_hbm.at[pl.ds(base, n_per_w)], idx_v)

        def chunk_body(kk, carry):
            cbase = kk * CH
            copies = []
            for j in range(n_sub):
                isl = idx_v.at[pl.ds(cbase + j * SG, SG)]
                copies.append(pltpu.async_copy(
                    w_hbm.at[isl], wbuf.at[pl.ds(j * SG, SG)], sem))
            for cp in copies:
                cp.wait()
            pltpu.sync_copy(wbuf, out_hbm.at[pl.ds(base + cbase, CH)])
            return carry

        lax.fori_loop(0, n_chunks, chunk_body, 0)

    return k


@functools.lru_cache(maxsize=None)
def _build_slow(n_total):
    n_per_w = n_total // NW
    CH = 640
    n_chunks = n_per_w // CH
    n_sub = CH // SG

    mesh = plsc.VectorSubcoreMesh(core_axis_name="c", subcore_axis_name="s")

    @functools.partial(
        pl.kernel,
        mesh=mesh,
        out_type=jax.ShapeDtypeStruct((n_total, D), jnp.float32),
        scratch_types=[
            pltpu.VMEM((n_per_w,), jnp.int32),
            pltpu.VMEM((CH, D), jnp.float32),
            pltpu.VMEM((CH, R), jnp.float32),
            pltpu.VMEM((R, D), jnp.float32),
            pltpu.SemaphoreType.DMA,
            pltpu.SemaphoreType.DMA,
        ],
        compiler_params=_SC_PARAMS,
    )
    def k(ids_hbm, w_hbm, a_hbm, b_hbm, out_hbm,
          idx_all, wbuf, bbuf, abuf, semw, semb):
        cid = lax.axis_index("c")
        sid = lax.axis_index("s")
        wid = sid * NC + cid
        base = wid * n_per_w
        pltpu.sync_copy(ids_hbm.at[pl.ds(base, n_per_w)], idx_all)
        pltpu.sync_copy(a_hbm, abuf)

        lane = lax.iota(jnp.int32, L)

        def chunk_body(kk, carry):
            cbase = kk * CH
            copies = []
            for j in range(n_sub):
                isl = idx_all.at[pl.ds(cbase + j * SG, SG)]
                copies.append(pltpu.async_copy(
                    w_hbm.at[isl], wbuf.at[pl.ds(j * SG, SG)], semw))
                copies.append(pltpu.async_copy(
                    b_hbm.at[isl], bbuf.at[pl.ds(j * SG, SG)], semb))
            for cp in copies:
                cp.wait()

            def row_body(rr, c2):
                full_r = jnp.full((L,), rr, jnp.int32)
                for c in range(D // L):
                    cols = c * L + lane
                    acc = plsc.load_gather(wbuf, [full_r, cols])
                    for r in range(R):
                        bv = plsc.load_gather(
                            bbuf, [full_r, jnp.full((L,), r, jnp.int32)])
                        av = abuf[r, pl.ds(c * L, L)]
                        acc = acc + (bv * SCALING) * av
                    plsc.store_scatter(wbuf, [full_r, cols], acc)
                return c2

            lax.fori_loop(0, CH, row_body, 0)

            pltpu.sync_copy(wbuf, out_hbm.at[pl.ds(base + cbase, CH)])
            return carry

        lax.fori_loop(0, n_chunks, chunk_body, 0)

    return k


def kernel(input_ids, weight, lora_A, lora_B):
    n_total = input_ids.shape[0] * input_ids.shape[1]
    ids = input_ids.reshape(n_total).astype(jnp.int32)

    any_nz = jnp.any(lora_B != 0)

    def fast():
        return _build_fast(n_total)(ids, weight)

    def slow():
        return _build_slow(n_total)(ids, weight, lora_A, lora_B)

    out = lax.cond(any_nz, slow, fast)
    return out.reshape(input_ids.shape + (D,))


# speedup vs baseline: 1.1826x; 1.0414x over previous
import functools

import jax
import jax.numpy as jnp
from jax import lax
from jax.experimental import pallas as pl
from jax.experimental.pallas import tpu as pltpu
from jax.experimental.pallas import tpu_sc as plsc

D = 64
R = 8
SCALING = 2.0
NC = 2
NS = 16
NW = NC * NS
L = 16

SG = 128

_SC_PARAMS = pltpu.CompilerParams(use_tc_tiling_on_sc=False,
                                  needs_layout_passes=False)


@functools.lru_cache(maxsize=None)
def _build_fast(n_total):
    n_per_w = n_total // NW
    CH = 640
    n_chunks = n_per_w // CH
    n_sub = CH // SG

    mesh = plsc.VectorSubcoreMesh(core_axis_name="c", subcore_axis_name="s")

    @functools.partial(
        pl.kernel,
        mesh=mesh,
        out_type=jax.ShapeDtypeStruct((n_total, D), jnp.float32),
        scratch_types=[
            pltpu.VMEM((n_per_w,), jnp.int32),
            pltpu.VMEM((CH, D), jnp.float32),
            pltpu.SemaphoreType.DMA,
        ],
        compiler_params=_SC_PARAMS,
    )
    def k(ids_hbm, w_hbm, out_hbm, idx_v, wbuf, sem):
        cid = lax.axis_index("c")
        sid = lax.axis_index("s")
        wid = sid * NC + cid
        base = wid * n_per_w
        pltpu.sync_copy(ids_hbm.at[pl.ds(base, n_per_w)], idx_v)

        def chunk_body(kk, carry):
            cbase = kk * CH
            copies = []
            for j in range(n_sub):
                isl = idx_v.at[pl.ds(cbase + j * SG, SG)]
                copies.append(pltpu.async_copy(
                    w_hbm.at[isl], wbuf.at[pl.ds(j * SG, SG)], sem))
            for cp in copies:
                cp.wait()
            pltpu.sync_copy(wbuf, out_hbm.at[pl.ds(base + cbase, CH)])
            return carry

        lax.fori_loop(0, n_chunks, chunk_body, 0)

    return k


@functools.lru_cache(maxsize=None)
def _build_slow(n_total):
    n_per_w = n_total // NW
    CH = 640
    n_chunks = n_per_w // CH
    n_sub = CH // SG

    mesh = plsc.VectorSubcoreMesh(core_axis_name="c", subcore_axis_name="s")

    @functools.partial(
        pl.kernel,
        mesh=mesh,
        out_type=jax.ShapeDtypeStruct((n_total, D), jnp.float32),
        scratch_types=[
            pltpu.VMEM((n_per_w,), jnp.int32),
            pltpu.VMEM((CH, D), jnp.float32),
            pltpu.VMEM((CH, R), jnp.float32),
            pltpu.VMEM((R, D), jnp.float32),
            pltpu.SemaphoreType.DMA,
            pltpu.SemaphoreType.DMA,
        ],
        compiler_params=_SC_PARAMS,
    )
    def k(ids_hbm, w_hbm, a_hbm, b_hbm, out_hbm,
          idx_all, wbuf, bbuf, abuf, semw, semb):
        cid = lax.axis_index("c")
        sid = lax.axis_index("s")
        wid = sid * NC + cid
        base = wid * n_per_w
        pltpu.sync_copy(ids_hbm.at[pl.ds(base, n_per_w)], idx_all)
        pltpu.sync_copy(a_hbm, abuf)

        lane = lax.iota(jnp.int32, L)

        def chunk_body(kk, carry):
            cbase = kk * CH
            copies = []
            for j in range(n_sub):
                isl = idx_all.at[pl.ds(cbase + j * SG, SG)]
                copies.append(pltpu.async_copy(
                    w_hbm.at[isl], wbuf.at[pl.ds(j * SG, SG)], semw))
                copies.append(pltpu.async_copy(
                    b_hbm.at[isl], bbuf.at[pl.ds(j * SG, SG)], semb))
            for cp in copies:
                cp.wait()

            def row_body(rr, c2):
                full_r = jnp.full((L,), rr, jnp.int32)
                for c in range(D // L):
                    cols = c * L + lane
                    acc = plsc.load_gather(wbuf, [full_r, cols])
                    for r in range(R):
                        bv = plsc.load_gather(
                            bbuf, [full_r, jnp.full((L,), r, jnp.int32)])
                        av = abuf[r, pl.ds(c * L, L)]
                        acc = acc + (bv * SCALING) * av
                    plsc.store_scatter(wbuf, [full_r, cols], acc)
                return c2

            lax.fori_loop(0, CH, row_body, 0)

            pltpu.sync_copy(wbuf, out_hbm.at[pl.ds(base + cbase, CH)])
            return carry

        lax.fori_loop(0, n_chunks, chunk_body, 0)

    return k


def kernel(input_ids, weight, lora_A, lora_B):
    n_total = input_ids.shape[0] * input_ids.shape[1]
    ids = input_ids.reshape(n_total).astype(jnp.int32)

    any_nz = jnp.any(lora_B != 0)
    base = _build_fast(n_total)(ids, weight)

    def slow():
        return _build_slow(n_total)(ids, weight, lora_A, lora_B)

    out = lax.cond(any_nz, slow, lambda: base)
    return out.reshape(input_ids.shape + (D,))
